# Initial kernel scaffold; baseline (speedup 1.0000x reference)
import jax
import jax.numpy as jnp
from jax.experimental import pallas as pl


def kernel(cur_embed, edge_weight, labels, alpha1, alpha2, alpha3, alpha4, edge_index):
    raise NotImplementedError("write your pallas kernel here")


# speedup vs baseline: 9.0151x; 9.0151x over previous
import functools

import jax
import jax.numpy as jnp
from jax import lax
from jax.experimental import pallas as pl
from jax.experimental.pallas import tpu as pltpu
from jax.experimental.pallas import tpu_sc as plsc

N = 100000
D = 32
E = 1600000
ITERS = 4
HALF = N // 2
PAD = 176
HPAD = HALF + PAD
NP = 2 * HPAD
TILE_N = HPAD // 16
UCH = 112
NCH = TILE_N // UCH
ROWB = 128
CH_E = 2048
PT = 100352
EPAD = 16 * PT
EROWS = EPAD // ROWB
TROWS = PT // ROWB
NSWEEP = PT // CH_E
SCH_E = 512
SROWS = SCH_E // ROWB
NSWEEP_S = PT // SCH_E

f32 = jnp.float32
i32 = jnp.int32

_mesh = plsc.VectorSubcoreMesh(core_axis_name="c", subcore_axis_name="s")
_cparams = pltpu.CompilerParams(use_tc_tiling_on_sc=False)


@functools.partial(
    pl.kernel,
    out_type=(
        jax.ShapeDtypeStruct((2, EROWS, ROWB), i32),
        jax.ShapeDtypeStruct((EROWS, ROWB), i32),
        jax.ShapeDtypeStruct((NP, D), f32),
    ),
    mesh=_mesh,
    compiler_params=_cparams,
    scratch_types=(
        pltpu.VMEM((16, ROWB), i32),
        pltpu.VMEM((16, ROWB), i32),
        pltpu.VMEM((16, ROWB), f32),
        pltpu.VMEM((16, ROWB), i32),
        pltpu.VMEM((16, ROWB), i32),
        pltpu.VMEM((16, ROWB), f32),
        pltpu.VMEM((UCH,), f32),
        pltpu.VMEM((UCH,), f32),
        pltpu.VMEM((UCH,), f32),
        pltpu.VMEM((UCH, D), f32),
        pltpu.VMEM((TILE_N,), f32),
        pltpu.VMEM((16,), f32),
        pltpu.VMEM_SHARED((HPAD,), f32),
    ),
)
def _prep(src2d, dst2d, w2d, labels_p, params, idx_out, srcadj_out, bias_out,
          dstv, srcv, wv, idxv, srcov, mv, ev, lv, bvv, rows, zb, pv, esum):
    c = lax.axis_index("c")
    s = lax.axis_index("s")
    pltpu.sync_copy(params, pv)
    pvec = pv[pl.ds(0, 16)]
    a2 = pvec[1]
    a3 = pvec[2]
    a4 = pvec[3]
    half_base = c * HALF
    trash = HALF + s
    z16 = jnp.zeros((16,), f32)
    for j in range(TILE_N // 16):
        zb[pl.ds(16 * j, 16)] = z16
    pltpu.sync_copy(zb, esum.at[pl.ds(s * TILE_N, TILE_N)])
    plsc.subcore_barrier()

    def sweep(i, carry):
        rowb = s * TROWS + i * 16
        pltpu.sync_copy(dst2d.at[pl.ds(rowb, 16), :], dstv)
        pltpu.sync_copy(w2d.at[pl.ds(rowb, 16), :], wv)
        pltpu.sync_copy(src2d.at[pl.ds(rowb, 16), :], srcv)
        for k in range(16):
            for l in range(8):
                sl = pl.ds(16 * l, 16)
                d = dstv[k, sl]
                off = d - half_base
                inb = (off >= 0) & (off < HALF)
                idxv[k, sl] = jnp.where(inb, off, trash)
                mv[k, sl] = jnp.maximum(a3 * wv[k, sl], 0.0)
                sr = srcv[k, sl]
                srcov[k, sl] = sr + jnp.where(sr >= HALF, PAD, 0)
        pltpu.sync_copy(idxv, idx_out.at[c, pl.ds(rowb, 16), :])

        @pl.when(c == 0)
        def _():
            pltpu.sync_copy(srcov, srcadj_out.at[pl.ds(rowb, 16), :])

        for k in range(16):
            pltpu.sync_copy(mv.at[k], esum.at[idxv.at[k]], add=True)
        return carry

    lax.fori_loop(0, NSWEEP, sweep, 0)
    plsc.subcore_barrier()

    def biasloop(ch, carry):
        nb = s * TILE_N + ch * UCH
        gbp = c * HPAD + nb
        pltpu.sync_copy(esum.at[pl.ds(nb, UCH)], ev)
        pltpu.sync_copy(labels_p.at[pl.ds(gbp, UCH)], lv)
        for j in range(UCH // 16):
            sl = pl.ds(16 * j, 16)
            bvv[sl] = a2 * ev[sl] + a4 * lv[sl]

        for g in range(UCH // 16):
            bvec = bvv[pl.ds(16 * g, 16)]
            for j in range(16):
                b = bvec[j]
                rows[16 * g + j, pl.ds(0, 16)] = jnp.broadcast_to(b, (16,))
                rows[16 * g + j, pl.ds(16, 16)] = jnp.broadcast_to(b, (16,))
        pltpu.sync_copy(rows, bias_out.at[pl.ds(gbp, UCH), :])
        return carry

    lax.fori_loop(0, NCH, biasloop, 0)


@functools.partial(
    pl.kernel,
    out_type=jax.ShapeDtypeStruct((NP, D), f32),
    mesh=_mesh,
    compiler_params=_cparams,
    scratch_types=(
        pltpu.VMEM((SROWS, ROWB), i32),
        pltpu.VMEM((SROWS, ROWB), i32),
        pltpu.VMEM((SCH_E, D), f32),
        pltpu.VMEM((UCH, D), f32),
        pltpu.VMEM((UCH, D), f32),
        pltpu.VMEM((UCH, D), f32),
        pltpu.VMEM((16,), f32),
        pltpu.VMEM_SHARED((HPAD, D), f32),
        pltpu.SemaphoreType.DMA,
        pltpu.SemaphoreType.DMA,
    ),
)
def _step(xp, srcadj, idx_all, bias_p, params, xout,
          srcv, idxv, rowsv, zrows, av, bv, pv, accum, sem, sem2):
    c = lax.axis_index("c")
    s = lax.axis_index("s")
    pltpu.sync_copy(params, pv)
    a1 = pv[pl.ds(0, 16)][0]
    z16 = jnp.zeros((16,), f32)
    for r in range(UCH):
        zrows[r, pl.ds(0, 16)] = z16
        zrows[r, pl.ds(16, 16)] = z16

    def zloop(i, carry):
        pltpu.sync_copy(zrows, accum.at[pl.ds(s * TILE_N + i * UCH, UCH), :])
        return carry

    lax.fori_loop(0, NCH, zloop, 0)
    plsc.subcore_barrier()

    def sweep(i, carry):
        rowb = s * TROWS + i * SROWS
        pltpu.sync_copy(srcadj.at[pl.ds(rowb, SROWS), :], srcv)
        pltpu.sync_copy(idx_all.at[c, pl.ds(rowb, SROWS), :], idxv)
        descs = [
            pltpu.async_copy(xp.at[srcv.at[k]],
                             rowsv.at[pl.ds(ROWB * k, ROWB), :], sem)
            for k in range(SROWS)
        ]
        for dd in descs:
            dd.wait()
        descs2 = [
            pltpu.async_copy(rowsv.at[pl.ds(ROWB * k, ROWB), :],
                             accum.at[idxv.at[k]], sem2, add=True)
            for k in range(SROWS)
        ]
        for dd in descs2:
            dd.wait()
        return carry

    lax.fori_loop(0, NSWEEP_S, sweep, 0)
    plsc.subcore_barrier()

    def upd(ch, carry):
        nb = s * TILE_N + ch * UCH
        gbp = c * HPAD + nb
        pltpu.sync_copy(accum.at[pl.ds(nb, UCH), :], av)
        pltpu.sync_copy(bias_p.at[pl.ds(gbp, UCH), :], bv)
        for r in range(UCH):
            for h in (0, 16):
                sl = pl.ds(h, 16)
                av[r, sl] = jnp.maximum(a1 * av[r, sl] + bv[r, sl], 0.0)
        pltpu.sync_copy(av, xout.at[pl.ds(gbp, UCH), :])
        return carry

    lax.fori_loop(0, NCH, upd, 0)


def kernel(cur_embed, edge_weight, labels, alpha1, alpha2, alpha3, alpha4, edge_index):
    src = edge_index[0].astype(i32)
    dst = edge_index[1].astype(i32)
    pe = EPAD - E
    pad_src = (jnp.arange(pe, dtype=i32) * 1009) % N
    src_p = jnp.concatenate([src, pad_src]).reshape(EROWS, ROWB)
    dst_p = jnp.concatenate([dst, jnp.full((pe,), N, i32)]).reshape(EROWS, ROWB)
    w_p = jnp.concatenate(
        [edge_weight.astype(f32), jnp.zeros((pe,), f32)]).reshape(EROWS, ROWB)
    zpad = jnp.zeros((PAD,), f32)
    labels_p = jnp.concatenate([labels[:HALF], zpad, labels[HALF:], zpad])
    zrows = jnp.zeros((PAD, D), f32)
    x = jnp.concatenate([cur_embed[:HALF], zrows, cur_embed[HALF:], zrows])
    params = jnp.concatenate(
        [alpha1, alpha2, alpha3, alpha4, jnp.zeros((12,), f32)]).astype(f32)
    idx_all, srcadj, bias_p = _prep(src_p, dst_p, w_p, labels_p, params)
    for _ in range(ITERS):
        x = _step(x, srcadj, idx_all, bias_p, params)
    return jnp.concatenate([x[:HALF], x[HPAD:HPAD + HALF]], axis=0)


# speedup vs baseline: 9.2446x; 1.0255x over previous
import functools

import jax
import jax.numpy as jnp
from jax import lax
from jax.experimental import pallas as pl
from jax.experimental.pallas import tpu as pltpu
from jax.experimental.pallas import tpu_sc as plsc

N = 100000
D = 32
E = 1600000
ITERS = 4
HALF = N // 2
PAD = 176
HPAD = HALF + PAD
NP = 2 * HPAD
TILE_N = HPAD // 16
UCH = 112
NCH = TILE_N // UCH
ROWB = 128
CH_E = 2048
PT = 100352
EPAD = 16 * PT
EROWS = EPAD // ROWB
TROWS = PT // ROWB
NSWEEP = PT // CH_E
SCH_E = 256
SROWS = SCH_E // ROWB
NSWEEP_S = PT // SCH_E
NS2 = NSWEEP_S // 2

f32 = jnp.float32
i32 = jnp.int32

_mesh = plsc.VectorSubcoreMesh(core_axis_name="c", subcore_axis_name="s")
_cparams = pltpu.CompilerParams(use_tc_tiling_on_sc=False)


@functools.partial(
    pl.kernel,
    out_type=(
        jax.ShapeDtypeStruct((2, EROWS, ROWB), i32),
        jax.ShapeDtypeStruct((EROWS, ROWB), i32),
        jax.ShapeDtypeStruct((NP, D), f32),
    ),
    mesh=_mesh,
    compiler_params=_cparams,
    scratch_types=(
        pltpu.VMEM((16, ROWB), i32),
        pltpu.VMEM((16, ROWB), i32),
        pltpu.VMEM((16, ROWB), f32),
        pltpu.VMEM((16, ROWB), i32),
        pltpu.VMEM((16, ROWB), i32),
        pltpu.VMEM((16, ROWB), f32),
        pltpu.VMEM((UCH,), f32),
        pltpu.VMEM((UCH,), f32),
        pltpu.VMEM((UCH,), f32),
        pltpu.VMEM((UCH, D), f32),
        pltpu.VMEM((TILE_N,), f32),
        pltpu.VMEM((16,), f32),
        pltpu.VMEM_SHARED((HPAD,), f32),
    ),
)
def _prep(src2d, dst2d, w2d, labels_p, params, idx_out, srcadj_out, bias_out,
          dstv, srcv, wv, idxv, srcov, mv, ev, lv, bvv, rows, zb, pv, esum):
    c = lax.axis_index("c")
    s = lax.axis_index("s")
    pltpu.sync_copy(params, pv)
    pvec = pv[pl.ds(0, 16)]
    a2 = pvec[1]
    a3 = pvec[2]
    a4 = pvec[3]
    half_base = c * HALF
    trash = HALF + s
    z16 = jnp.zeros((16,), f32)
    for j in range(TILE_N // 16):
        zb[pl.ds(16 * j, 16)] = z16
    pltpu.sync_copy(zb, esum.at[pl.ds(s * TILE_N, TILE_N)])
    plsc.subcore_barrier()

    def sweep(i, carry):
        rowb = s * TROWS + i * 16
        pltpu.sync_copy(dst2d.at[pl.ds(rowb, 16), :], dstv)
        pltpu.sync_copy(w2d.at[pl.ds(rowb, 16), :], wv)
        pltpu.sync_copy(src2d.at[pl.ds(rowb, 16), :], srcv)
        for k in range(16):
            for l in range(8):
                sl = pl.ds(16 * l, 16)
                d = dstv[k, sl]
                off = d - half_base
                inb = (off >= 0) & (off < HALF)
                idxv[k, sl] = jnp.where(inb, off, trash)
                mv[k, sl] = jnp.maximum(a3 * wv[k, sl], 0.0)
                sr = srcv[k, sl]
                srcov[k, sl] = sr + jnp.where(sr >= HALF, PAD, 0)
        pltpu.sync_copy(idxv, idx_out.at[c, pl.ds(rowb, 16), :])

        @pl.when(c == 0)
        def _():
            pltpu.sync_copy(srcov, srcadj_out.at[pl.ds(rowb, 16), :])

        for k in range(16):
            pltpu.sync_copy(mv.at[k], esum.at[idxv.at[k]], add=True)
        return carry

    lax.fori_loop(0, NSWEEP, sweep, 0)
    plsc.subcore_barrier()

    def biasloop(ch, carry):
        nb = s * TILE_N + ch * UCH
        gbp = c * HPAD + nb
        pltpu.sync_copy(esum.at[pl.ds(nb, UCH)], ev)
        pltpu.sync_copy(labels_p.at[pl.ds(gbp, UCH)], lv)
        for j in range(UCH // 16):
            sl = pl.ds(16 * j, 16)
            bvv[sl] = a2 * ev[sl] + a4 * lv[sl]

        for g in range(UCH // 16):
            bvec = bvv[pl.ds(16 * g, 16)]
            for j in range(16):
                b = bvec[j]
                rows[16 * g + j, pl.ds(0, 16)] = jnp.broadcast_to(b, (16,))
                rows[16 * g + j, pl.ds(16, 16)] = jnp.broadcast_to(b, (16,))
        pltpu.sync_copy(rows, bias_out.at[pl.ds(gbp, UCH), :])
        return carry

    lax.fori_loop(0, NCH, biasloop, 0)


@functools.partial(
    pl.kernel,
    out_type=jax.ShapeDtypeStruct((NP, D), f32),
    mesh=_mesh,
    compiler_params=_cparams,
    scratch_types=(
        pltpu.VMEM((SROWS, ROWB), i32),
        pltpu.VMEM((SROWS, ROWB), i32),
        pltpu.VMEM((SROWS, ROWB), i32),
        pltpu.VMEM((SROWS, ROWB), i32),
        pltpu.VMEM((SCH_E, D), f32),
        pltpu.VMEM((SCH_E, D), f32),
        pltpu.VMEM((UCH, D), f32),
        pltpu.VMEM((UCH, D), f32),
        pltpu.VMEM((UCH, D), f32),
        pltpu.VMEM((16,), f32),
        pltpu.VMEM_SHARED((HPAD, D), f32),
        pltpu.SemaphoreType.DMA,
        pltpu.SemaphoreType.DMA,
    ),
)
def _step(xp, srcadj, idx_all, bias_p, params, xout,
          srcva, srcvb, idxva, idxvb, rowsa, rowsb, zrows, av, bv, pv,
          accum, sema, semb):
    c = lax.axis_index("c")
    s = lax.axis_index("s")
    pltpu.sync_copy(params, pv)
    a1 = pv[pl.ds(0, 16)][0]
    z16 = jnp.zeros((16,), f32)
    for r in range(UCH):
        zrows[r, pl.ds(0, 16)] = z16
        zrows[r, pl.ds(16, 16)] = z16

    def zloop(i, carry):
        pltpu.sync_copy(zrows, accum.at[pl.ds(s * TILE_N + i * UCH, UCH), :])
        return carry

    lax.fori_loop(0, NCH, zloop, 0)
    plsc.subcore_barrier()

    def fire(ch, srcb, idxb, rowsb_, gsem):
        rowb = s * TROWS + ch * SROWS
        pltpu.sync_copy(srcadj.at[pl.ds(rowb, SROWS), :], srcb)
        pltpu.sync_copy(idx_all.at[c, pl.ds(rowb, SROWS), :], idxb)
        for k in range(SROWS):
            pltpu.async_copy(xp.at[srcb.at[k]],
                             rowsb_.at[pl.ds(ROWB * k, ROWB), :], gsem)

    def drain(rowsb_, gsem):
        for k in range(SROWS):
            pltpu.make_async_copy(
                xp.at[pl.ds(0, ROWB), :],
                rowsb_.at[pl.ds(ROWB * k, ROWB), :], gsem).wait()

    def scat(idxb, rowsb_):
        for k in range(SROWS):
            pltpu.sync_copy(rowsb_.at[pl.ds(ROWB * k, ROWB), :],
                            accum.at[idxb.at[k]], add=True)

    fire(0, srcva, idxva, rowsa, sema)

    def sweep(j, carry):
        fire(2 * j + 1, srcvb, idxvb, rowsb, semb)
        drain(rowsa, sema)
        scat(idxva, rowsa)

        @pl.when(j < NS2 - 1)
        def _():
            fire(2 * j + 2, srcva, idxva, rowsa, sema)

        drain(rowsb, semb)
        scat(idxvb, rowsb)
        return carry

    lax.fori_loop(0, NS2, sweep, 0)
    plsc.subcore_barrier()

    def upd(ch, carry):
        nb = s * TILE_N + ch * UCH
        gbp = c * HPAD + nb
        pltpu.sync_copy(accum.at[pl.ds(nb, UCH), :], av)
        pltpu.sync_copy(bias_p.at[pl.ds(gbp, UCH), :], bv)
        for r in range(UCH):
            for h in (0, 16):
                sl = pl.ds(h, 16)
                av[r, sl] = jnp.maximum(a1 * av[r, sl] + bv[r, sl], 0.0)
        pltpu.sync_copy(av, xout.at[pl.ds(gbp, UCH), :])
        return carry

    lax.fori_loop(0, NCH, upd, 0)


def kernel(cur_embed, edge_weight, labels, alpha1, alpha2, alpha3, alpha4, edge_index):
    src = edge_index[0].astype(i32)
    dst = edge_index[1].astype(i32)
    pe = EPAD - E
    pad_src = (jnp.arange(pe, dtype=i32) * 1009) % N
    src_p = jnp.concatenate([src, pad_src]).reshape(EROWS, ROWB)
    dst_p = jnp.concatenate([dst, jnp.full((pe,), N, i32)]).reshape(EROWS, ROWB)
    w_p = jnp.concatenate(
        [edge_weight.astype(f32), jnp.zeros((pe,), f32)]).reshape(EROWS, ROWB)
    zpad = jnp.zeros((PAD,), f32)
    labels_p = jnp.concatenate([labels[:HALF], zpad, labels[HALF:], zpad])
    zrows = jnp.zeros((PAD, D), f32)
    x = jnp.concatenate([cur_embed[:HALF], zrows, cur_embed[HALF:], zrows])
    params = jnp.concatenate(
        [alpha1, alpha2, alpha3, alpha4, jnp.zeros((12,), f32)]).astype(f32)
    idx_all, srcadj, bias_p = _prep(src_p, dst_p, w_p, labels_p, params)
    for _ in range(ITERS):
        x = _step(x, srcadj, idx_all, bias_p, params)
    return jnp.concatenate([x[:HALF], x[HPAD:HPAD + HALF]], axis=0)
